# baseline (speedup 1.0000x reference)
import jax
import jax.numpy as jnp
from jax import lax
from jax.experimental import pallas as pl
from jax.experimental.pallas import tpu as pltpu
from jax.experimental.pallas import tpu_sc as plsc

N_NODES = 10000
N_EDGES = 320000
HID = 128

NC = 2
NS = 16
NW = NC * NS
E_PER = N_EDGES // NW
L = 16

BR = 5120
GRID = (N_NODES + BR - 1) // BR
N_PAD = GRID * BR

ALEN = (E_PER // 128 + 1) * 128

UNROLL = 16


def _sc_hist_body(edge_hbm, out_hbm, idx_v, hist_v, sem):
    c = lax.axis_index("c")
    s = lax.axis_index("s")
    wid = s * NC + c
    start = wid * E_PER
    base_al = start // 128 * 128
    off = start - base_al
    cp = pltpu.make_async_copy(
        edge_hbm.at[:, pl.ds(base_al, ALEN)], idx_v, sem)
    cp.start()

    zeros = jnp.zeros((L,), jnp.float32)

    def zbody(i, carry):
        for j in range(UNROLL):
            hist_v[pl.ds((i * UNROLL + j) * L, L)] = zeros
        return carry

    lax.fori_loop(0, N_PAD // L // UNROLL, zbody, 0)
    cp.wait()

    ones = jnp.ones((L,), jnp.float32)

    def body(i, carry):
        for j in range(UNROLL):
            idx = idx_v[0, pl.ds(off + (i * UNROLL + j) * L, L)]
            plsc.addupdate_scatter(hist_v, [idx], ones)
        return carry

    lax.fori_loop(0, E_PER // L // UNROLL, body, 0)
    for j in range(E_PER // L - (E_PER // L // UNROLL) * UNROLL):
        base = ((E_PER // L // UNROLL) * UNROLL + j) * L
        idx = idx_v[0, pl.ds(off + base, L)]
        plsc.addupdate_scatter(hist_v, [idx], ones)

    pltpu.sync_copy(hist_v, out_hbm.at[wid])


def _sc_hist(edge_index):
    mesh = plsc.VectorSubcoreMesh(core_axis_name="c", subcore_axis_name="s")
    return pl.kernel(
        _sc_hist_body,
        out_type=jax.ShapeDtypeStruct((NW, N_PAD), jnp.float32),
        mesh=mesh,
        compiler_params=pltpu.CompilerParams(needs_layout_passes=False),
        scratch_types=[
            pltpu.VMEM((2, ALEN), jnp.int32),
            pltpu.VMEM((N_PAD,), jnp.float32),
            pltpu.SemaphoreType.DMA,
        ],
    )(edge_index)


def _tc_main_body(pf_ref, x_ref, w_ref, b_ref, o_ref, m_sc):
    pid = pl.program_id(0)

    @pl.when(pid == 0)
    def _():
        deg = jnp.sum(pf_ref[...], axis=0, keepdims=True)
        m = jnp.max(deg)
        m_sc[0, 0] = 1.0 / (m + 1e-8)

    inv = m_sc[0, 0]
    p = pf_ref[:, pl.ds(pid * BR, BR)]
    dn = jnp.sum(p, axis=0, keepdims=True) * inv
    iota = lax.broadcasted_iota(jnp.int32, (1, BR), 1)
    idxn = (iota + pid * BR).astype(jnp.float32) * (1.0 / (N_NODES - 1))
    rw = jnp.sqrt(dn + 1e-8)
    f = jnp.concatenate([dn, idxn, rw], axis=0)
    pos = lax.dot_general(
        f, w_ref[...],
        (((0,), (1,)), ((), ())),
        preferred_element_type=jnp.float32,
        precision=lax.Precision.HIGHEST,
    )
    o_ref[...] = x_ref[...] + pos + b_ref[...]


def _tc_main(partials, x, W, b_row):
    return pl.pallas_call(
        _tc_main_body,
        grid=(GRID,),
        in_specs=[
            pl.BlockSpec((NW, N_PAD), lambda i: (0, 0)),
            pl.BlockSpec((BR, HID), lambda i: (i, 0)),
            pl.BlockSpec((HID, 3), lambda i: (0, 0)),
            pl.BlockSpec((1, HID), lambda i: (0, 0)),
        ],
        out_specs=pl.BlockSpec((BR, HID), lambda i: (i, 0)),
        out_shape=jax.ShapeDtypeStruct((N_NODES, HID), jnp.float32),
        scratch_shapes=[pltpu.SMEM((1, 1), jnp.float32)],
    )(partials, x, W, b_row)


@jax.jit
def kernel(x, edge_index, batch, W, b):
    del batch
    partials = _sc_hist(edge_index)
    return _tc_main(partials, x, W, b[None, :])


# speedup vs baseline: 1.0056x; 1.0056x over previous
import jax
import jax.numpy as jnp
from jax import lax
from jax.experimental import pallas as pl
from jax.experimental.pallas import tpu as pltpu
from jax.experimental.pallas import tpu_sc as plsc

N_NODES = 10000
N_EDGES = 320000
HID = 128

NC = 2
NS = 16
NW = NC * NS
E_PER = N_EDGES // NW
L = 16

BR = 5120
GRID = (N_NODES + BR - 1) // BR
N_PAD = GRID * BR

ALEN = (E_PER // 128 + 1) * 128

UNROLL = 8


def _sc_hist_body(edge_hbm, out_hbm, idx_v, hist_v, sem):
    c = lax.axis_index("c")
    s = lax.axis_index("s")
    wid = s * NC + c
    start = wid * E_PER
    base_al = start // 128 * 128
    off = start - base_al
    cp = pltpu.make_async_copy(
        edge_hbm.at[:, pl.ds(base_al, ALEN)], idx_v, sem)
    cp.start()

    zeros = jnp.zeros((L,), jnp.float32)

    def zbody(i, carry):
        for j in range(UNROLL):
            hist_v[pl.ds((i * UNROLL + j) * L, L)] = zeros
        return carry

    lax.fori_loop(0, N_PAD // L // UNROLL, zbody, 0)
    cp.wait()

    ones = jnp.ones((L,), jnp.float32)

    def body(i, carry):
        for j in range(UNROLL):
            idx = idx_v[0, pl.ds(off + (i * UNROLL + j) * L, L)]
            plsc.addupdate_scatter(hist_v, [idx], ones)
        return carry

    lax.fori_loop(0, E_PER // L // UNROLL, body, 0)
    for j in range(E_PER // L - (E_PER // L // UNROLL) * UNROLL):
        base = ((E_PER // L // UNROLL) * UNROLL + j) * L
        idx = idx_v[0, pl.ds(off + base, L)]
        plsc.addupdate_scatter(hist_v, [idx], ones)

    pltpu.sync_copy(hist_v, out_hbm.at[wid])


def _sc_hist(edge_index):
    mesh = plsc.VectorSubcoreMesh(core_axis_name="c", subcore_axis_name="s")
    return pl.kernel(
        _sc_hist_body,
        out_type=jax.ShapeDtypeStruct((NW, N_PAD), jnp.float32),
        mesh=mesh,
        compiler_params=pltpu.CompilerParams(needs_layout_passes=False),
        scratch_types=[
            pltpu.VMEM((2, ALEN), jnp.int32),
            pltpu.VMEM((N_PAD,), jnp.float32),
            pltpu.SemaphoreType.DMA,
        ],
    )(edge_index)


def _tc_main_body(pf_ref, x_ref, w_ref, b_ref, o_ref, m_sc):
    pid = pl.program_id(0)

    @pl.when(pid == 0)
    def _():
        deg = jnp.sum(pf_ref[...], axis=0, keepdims=True)
        m = jnp.max(deg)
        m_sc[0, 0] = 1.0 / (m + 1e-8)

    inv = m_sc[0, 0]
    p = pf_ref[:, pl.ds(pid * BR, BR)]
    dn = jnp.sum(p, axis=0, keepdims=True) * inv
    iota = lax.broadcasted_iota(jnp.int32, (1, BR), 1)
    idxn = (iota + pid * BR).astype(jnp.float32) * (1.0 / (N_NODES - 1))
    rw = jnp.sqrt(dn + 1e-8)
    f = jnp.concatenate([dn, idxn, rw], axis=0)
    pos = lax.dot_general(
        f, w_ref[...],
        (((0,), (1,)), ((), ())),
        preferred_element_type=jnp.float32,
        precision=lax.Precision.HIGHEST,
    )
    o_ref[...] = x_ref[...] + pos + b_ref[...]


def _tc_main(partials, x, W, b_row):
    return pl.pallas_call(
        _tc_main_body,
        grid=(GRID,),
        in_specs=[
            pl.BlockSpec((NW, N_PAD), lambda i: (0, 0)),
            pl.BlockSpec((BR, HID), lambda i: (i, 0)),
            pl.BlockSpec((HID, 3), lambda i: (0, 0)),
            pl.BlockSpec((1, HID), lambda i: (0, 0)),
        ],
        out_specs=pl.BlockSpec((BR, HID), lambda i: (i, 0)),
        out_shape=jax.ShapeDtypeStruct((N_NODES, HID), jnp.float32),
        scratch_shapes=[pltpu.SMEM((1, 1), jnp.float32)],
    )(partials, x, W, b_row)


@jax.jit
def kernel(x, edge_index, batch, W, b):
    del batch
    partials = _sc_hist(edge_index)
    return _tc_main(partials, x, W, b[None, :])
